# Initial kernel scaffold; baseline (speedup 1.0000x reference)
import jax
import jax.numpy as jnp
from jax.experimental import pallas as pl


def kernel(ln_gamma, ln_beta, Wq, bq, Wk, bk, Wv, bv, Wp, bp, Wg1, bg1, Wg2, bg2, adj1, adj2, x):
    raise NotImplementedError("write your pallas kernel here")


# speedup vs baseline: 5.6317x; 5.6317x over previous
import math

import jax
import jax.numpy as jnp
from jax import lax
from jax.experimental import pallas as pl
from jax.experimental.pallas import tpu as pltpu

_F32 = jnp.float32
_BF16 = jnp.bfloat16


def _adj_norm_kernel(a_ref, l_ref):
    a = jnp.maximum(a_ref[0].astype(_F32), 0.0)
    deg = jnp.sum(a, axis=1, keepdims=True)
    dinv = lax.rsqrt(deg + 1e-10)
    outer = lax.dot_general(dinv, dinv, (((1,), (1,)), ((), ())),
                            preferred_element_type=_F32)
    l_ref[0] = a * outer


def _block_kernel(x_ref, gamma_ref, beta_ref, wqkv_ref, bqkv_ref,
                  wp_ref, bp_ref, wg_ref, bg_ref, l_ref, out_ref):
    x = x_ref[0].astype(_F32)
    n, emb = x.shape
    num_heads = 8
    d = emb // num_heads
    half = num_heads // 2
    inv_scale = 1.0 / math.sqrt(float(emb))

    mu = jnp.mean(x, axis=-1, keepdims=True)
    xc = x - mu
    var = jnp.mean(xc * xc, axis=-1, keepdims=True)
    xn = xc * lax.rsqrt(var + 1e-5)
    xn = xn * gamma_ref[...] + beta_ref[...]

    z = jnp.dot(xn.astype(_BF16), wqkv_ref[...],
                preferred_element_type=_F32) + bqkv_ref[...]
    q = z[:, :emb]
    k = z[:, emb:2 * emb]
    v = z[:, 2 * emb:]

    hd = half * d
    g1 = jnp.sum(q[:, :hd] * wg_ref[0], axis=-1, keepdims=True)
    g2 = jnp.sum(q[:, hd:] * wg_ref[1], axis=-1, keepdims=True)
    w1 = jax.nn.sigmoid(g1 + bg_ref[0, 0])
    w2 = jax.nn.sigmoid(g2 + bg_ref[0, 1])

    qb = (q * inv_scale).astype(_BF16)
    kb = k.astype(_BF16)
    vb = v.astype(_BF16)
    wl1 = (w1 * inv_scale) * l_ref[0]
    wl2 = (w2 * inv_scale) * l_ref[1]

    ohs = []
    for h in range(num_heads):
        sl = slice(h * d, (h + 1) * d)
        e = lax.dot_general(qb[:, sl], kb[:, sl], (((1,), (1,)), ((), ())),
                            preferred_element_type=_F32)
        e = e + (wl1 if h < half else wl2)
        m = jnp.max(e, axis=-1, keepdims=True)
        p = jnp.exp(e - m)
        att = (p / jnp.sum(p, axis=-1, keepdims=True)).astype(_BF16)
        ohs.append(jnp.dot(att, vb[:, sl],
                           preferred_element_type=_F32).astype(_BF16))

    o = jnp.concatenate(ohs, axis=1)
    acc = jnp.dot(o, wp_ref[...], preferred_element_type=_F32)
    out_ref[0] = (acc + bp_ref[...] + x).astype(out_ref.dtype)


def kernel(ln_gamma, ln_beta, Wq, bq, Wk, bk, Wv, bv, Wp, bp,
           Wg1, bg1, Wg2, bg2, adj1, adj2, x):
    B, n, emb = x.shape

    adj = jnp.stack([adj1, adj2]).astype(_F32)
    l_stack = pl.pallas_call(
        _adj_norm_kernel,
        out_shape=jax.ShapeDtypeStruct((2, n, n), _F32),
        grid=(2,),
        in_specs=[pl.BlockSpec((1, n, n), lambda i: (i, 0, 0))],
        out_specs=pl.BlockSpec((1, n, n), lambda i: (i, 0, 0)),
        compiler_params=pltpu.CompilerParams(
            dimension_semantics=("parallel",)),
    )(adj)

    wqkv = jnp.concatenate(
        [jnp.transpose(Wq), jnp.transpose(Wk), jnp.transpose(Wv)],
        axis=1).astype(_BF16)
    bqkv = jnp.concatenate([bq, bk, bv]).reshape(1, 3 * emb).astype(_F32)
    wp_t = jnp.transpose(Wp).astype(_BF16)
    bp_r = bp.reshape(1, emb).astype(_F32)
    hd = emb // 2
    wg = jnp.stack([Wg1.reshape(hd), Wg2.reshape(hd)]).astype(_F32)
    bg = jnp.stack([bg1.reshape(()), bg2.reshape(())]).reshape(1, 2).astype(_F32)
    gamma = ln_gamma.reshape(1, emb).astype(_F32)
    beta = ln_beta.reshape(1, emb).astype(_F32)

    def full_spec(shape):
        nd = len(shape)
        return pl.BlockSpec(shape, lambda b, _nd=nd: (0,) * _nd)

    return pl.pallas_call(
        _block_kernel,
        out_shape=jax.ShapeDtypeStruct((B, n, emb), x.dtype),
        grid=(B,),
        in_specs=[
            pl.BlockSpec((1, n, emb), lambda b: (b, 0, 0)),
            full_spec((1, emb)), full_spec((1, emb)),
            full_spec((emb, 3 * emb)), full_spec((1, 3 * emb)),
            full_spec((emb, emb)), full_spec((1, emb)),
            full_spec((2, hd)), full_spec((1, 2)),
            full_spec((2, n, n)),
        ],
        out_specs=pl.BlockSpec((1, n, emb), lambda b: (b, 0, 0)),
        compiler_params=pltpu.CompilerParams(
            dimension_semantics=("parallel",),
            vmem_limit_bytes=64 << 20,
        ),
    )(x, gamma, beta, wqkv, bqkv, wp_t, bp_r, wg, bg, l_stack)


# speedup vs baseline: 8.5907x; 1.5254x over previous
import functools
import math

import jax
import jax.numpy as jnp
from jax import lax
from jax.experimental import pallas as pl
from jax.experimental.pallas import tpu as pltpu

_F32 = jnp.float32
_BF16 = jnp.bfloat16
_H = 8


def _norm_adj(a, scale):
    a = jnp.maximum(a.astype(_F32), 0.0)
    deg = jnp.sum(a, axis=1, keepdims=True)
    dinv = lax.rsqrt(deg + 1e-10)
    outer = lax.dot_general(dinv, dinv * scale, (((1,), (1,)), ((), ())),
                            preferred_element_type=_F32)
    return a * outer


def _prep_kernel(wq_ref, wk_ref, wv_ref, wp_ref,
                 bq_ref, bk_ref, bv_ref,
                 wg1_ref, wg2_ref, bg1_ref, bg2_ref,
                 a1_ref, a2_ref,
                 wqkv_ref, bqkv_ref, wpb_ref, l_ref, *, scale):
    emb = wq_ref.shape[0]
    d = emb // _H
    hd = emb // 2
    vw = d + 1

    wqkv_ref[0:emb, :] = (wq_ref[...] * scale).astype(_BF16)
    wqkv_ref[emb:2 * emb, :] = wk_ref[...].astype(_BF16)
    bqkv_ref[0:1, 0:emb] = bq_ref[...] * scale
    bqkv_ref[0:1, emb:2 * emb] = bk_ref[...]
    for h in range(_H):
        base = 2 * emb + h * vw
        wqkv_ref[base:base + d, :] = wv_ref[h * d:(h + 1) * d, :].astype(_BF16)
        wqkv_ref[base + d:base + d + 1, :] = jnp.zeros((1, emb), _BF16)
        bqkv_ref[0:1, base:base + d] = bv_ref[0:1, h * d:(h + 1) * d]
        bqkv_ref[0:1, base + d:base + d + 1] = jnp.ones((1, 1), _F32)
    goff = 2 * emb + _H * vw
    g1row = lax.dot_general(wg1_ref[...], wq_ref[0:hd, :],
                            (((1,), (0,)), ((), ())),
                            preferred_element_type=_F32)
    g2row = lax.dot_general(wg2_ref[...], wq_ref[hd:emb, :],
                            (((1,), (0,)), ((), ())),
                            preferred_element_type=_F32)
    wqkv_ref[goff:goff + 1, :] = g1row.astype(_BF16)
    wqkv_ref[goff + 1:goff + 2, :] = g2row.astype(_BF16)
    gb1 = (jnp.sum(bq_ref[0:1, 0:hd] * wg1_ref[...], axis=1, keepdims=True)
           + bg1_ref[...])
    gb2 = (jnp.sum(bq_ref[0:1, hd:emb] * wg2_ref[...], axis=1, keepdims=True)
           + bg2_ref[...])
    bqkv_ref[0:1, goff:goff + 1] = gb1
    bqkv_ref[0:1, goff + 1:goff + 2] = gb2

    wpb_ref[...] = wp_ref[...].astype(_BF16)
    l_ref[0] = _norm_adj(a1_ref[...], scale)
    l_ref[1] = _norm_adj(a2_ref[...], scale)


def _block_kernel(x_ref, gamma_ref, beta_ref, wqkv_ref, bqkv_ref,
                  wp_ref, bp_ref, l_ref, out_ref):
    _, n, emb = x_ref.shape
    d = emb // _H
    half = _H // 2
    vw = d + 1
    goff = 2 * emb + _H * vw

    x = x_ref[0].astype(_F32)

    mu = jnp.mean(x, axis=-1, keepdims=True)
    xc = x - mu
    var = jnp.mean(xc * xc, axis=-1, keepdims=True)
    xn = xc * lax.rsqrt(var + 1e-5)
    xn = xn * gamma_ref[...] + beta_ref[...]

    z = lax.dot_general(xn.astype(_BF16), wqkv_ref[...],
                        (((1,), (1,)), ((), ())),
                        preferred_element_type=_F32) + bqkv_ref[...]
    w1 = jax.nn.sigmoid(z[:, goff:goff + 1])
    w2 = jax.nn.sigmoid(z[:, goff + 1:goff + 2])

    zb = z.astype(_BF16)
    qb = zb[:, :emb]
    kb = zb[:, emb:2 * emb]
    vab = zb[:, 2 * emb:goff]
    wl1 = w1 * l_ref[0]
    wl2 = w2 * l_ref[1]

    ohs = []
    for h in range(_H):
        cols = slice(h * d, (h + 1) * d)
        en = lax.dot_general(qb[:, cols], kb[:, cols],
                             (((1,), (1,)), ((), ())),
                             preferred_element_type=_F32)
        en = en + (wl1 if h < half else wl2)
        m = jnp.max(en, axis=-1, keepdims=True)
        p = jnp.exp(en - m).astype(_BF16)
        osum = jnp.dot(p, vab[:, h * vw:(h + 1) * vw],
                       preferred_element_type=_F32)
        ohs.append((osum[:, :d] * (1.0 / osum[:, d:])).astype(_BF16))

    o = jnp.concatenate(ohs, axis=1)
    acc = lax.dot_general(o, wp_ref[...], (((1,), (1,)), ((), ())),
                          preferred_element_type=_F32)
    out_ref[0] = (acc + bp_ref[...] + x).astype(out_ref.dtype)


def kernel(ln_gamma, ln_beta, Wq, bq, Wk, bk, Wv, bv, Wp, bp,
           Wg1, bg1, Wg2, bg2, adj1, adj2, x):
    B, n, emb = x.shape
    d = emb // _H
    hd = emb // 2
    vw = d + 1
    zw = 2 * emb + _H * vw + 2
    inv_scale = 1.0 / math.sqrt(float(emb))

    def full_spec(shape):
        nd = len(shape)
        return pl.BlockSpec(shape, lambda b, _nd=nd: (0,) * _nd)

    wqkv, bqkv, wpb, l_stack = pl.pallas_call(
        functools.partial(_prep_kernel, scale=inv_scale),
        out_shape=(jax.ShapeDtypeStruct((zw, emb), _BF16),
                   jax.ShapeDtypeStruct((1, zw), _F32),
                   jax.ShapeDtypeStruct((emb, emb), _BF16),
                   jax.ShapeDtypeStruct((2, n, n), _F32)),
        grid=(1,),
        in_specs=[full_spec((emb, emb))] * 4 + [full_spec((1, emb))] * 3 +
                 [full_spec((1, hd))] * 2 + [full_spec((1, 1))] * 2 +
                 [full_spec((n, n))] * 2,
        out_specs=(full_spec((zw, emb)), full_spec((1, zw)),
                   full_spec((emb, emb)), full_spec((2, n, n))),
    )(Wq, Wk, Wv, Wp,
      bq.reshape(1, emb), bk.reshape(1, emb), bv.reshape(1, emb),
      Wg1.reshape(1, hd), Wg2.reshape(1, hd),
      bg1.reshape(1, 1), bg2.reshape(1, 1),
      adj1, adj2)

    gamma = ln_gamma.reshape(1, emb)
    beta = ln_beta.reshape(1, emb)
    bp_r = bp.reshape(1, emb)

    return pl.pallas_call(
        _block_kernel,
        out_shape=jax.ShapeDtypeStruct((B, n, emb), x.dtype),
        grid=(B,),
        in_specs=[
            pl.BlockSpec((1, n, emb), lambda b: (b, 0, 0)),
            full_spec((1, emb)), full_spec((1, emb)),
            full_spec((zw, emb)), full_spec((1, zw)),
            full_spec((emb, emb)), full_spec((1, emb)),
            full_spec((2, n, n)),
        ],
        out_specs=pl.BlockSpec((1, n, emb), lambda b: (b, 0, 0)),
        compiler_params=pltpu.CompilerParams(
            dimension_semantics=("parallel",),
            vmem_limit_bytes=64 << 20,
        ),
    )(x, gamma, beta, wqkv, bqkv, wpb, bp_r, l_stack)
